# Initial kernel scaffold; baseline (speedup 1.0000x reference)
import jax
import jax.numpy as jnp
from jax.experimental import pallas as pl


def kernel(x, adj, e, W):
    raise NotImplementedError("write your pallas kernel here")


# speedup vs baseline: 9.6190x; 9.6190x over previous
import functools

import jax
import jax.numpy as jnp
from jax import lax
from jax.experimental import pallas as pl
from jax.experimental.pallas import tpu as pltpu
from jax.experimental.pallas import tpu_sc as plsc

N = 10000
E = 320000
F = 128
K = 4

NC = 2
NS = 16
NW = NC * NS
L = 16

EPW = E // NW
DN = 10112
B = 64
NCHUNK = 160
CPW = NCHUNK * B
EPAD = NW * CPW
ZR = 632
NPAD = NS * ZR
CR = 624

_mesh = plsc.VectorSubcoreMesh(core_axis_name="c", subcore_axis_name="s")


_DCH = 16


@functools.partial(
    pl.kernel,
    out_type=jax.ShapeDtypeStruct((NW, 1, DN), jnp.float32),
    mesh=_mesh,
    scratch_types=[
        pltpu.VMEM((_DCH, B), jnp.int32),
        pltpu.VMEM((1, DN), jnp.float32),
    ],
    compiler_params=pltpu.CompilerParams(needs_layout_passes=False),
)
def _sc_degree(col_hbm, dpart_hbm, colv, dloc):
    wid = lax.axis_index("s") * NC + lax.axis_index("c")

    def _zero(i, _):
        dloc[0, pl.ds(i * L, L)] = jnp.zeros((L,), jnp.float32)
        return _

    lax.fori_loop(0, DN // L, _zero, None)
    ones = jnp.ones((L,), jnp.float32)
    zrow = jnp.zeros((L,), jnp.int32)
    vpr = B // L

    def _acc(i, _):
        idx = colv[i // vpr, pl.ds((i % vpr) * L, L)]
        plsc.addupdate_scatter(dloc, [zrow, idx], ones)
        return _

    for cc in range(NCHUNK // _DCH):
        pltpu.sync_copy(col_hbm.at[wid, pl.ds(cc * _DCH, _DCH), :], colv)
        lax.fori_loop(0, _DCH * vpr, _acc, None)
    pltpu.sync_copy(dloc, dpart_hbm.at[wid])


@functools.partial(
    pl.kernel,
    out_type=jax.ShapeDtypeStruct((NC, N, F), jnp.float32),
    mesh=_mesh,
    scratch_types=[
        pltpu.VMEM((2, 2, B), jnp.int32),
        pltpu.VMEM((2, B, F), jnp.float32),
        pltpu.VMEM_SHARED((NPAD, F), jnp.float32),
        pltpu.SemaphoreType.DMA,
        pltpu.SemaphoreType.DMA,
        pltpu.SemaphoreType.DMA,
        pltpu.SemaphoreType.DMA,
    ],
    compiler_params=pltpu.CompilerParams(needs_layout_passes=False),
)
def _sc_spmm(rc_hbm, xs_hbm, aggp_hbm, rcv, buf, agg, si0, si1, sg0, sg1):
    cid = lax.axis_index("c")
    sid = lax.axis_index("s")
    wid = sid * NC + cid

    def _zrow(r, _):
        for c in range(F // L):
            buf[0, r, pl.ds(c * L, L)] = jnp.zeros((L,), jnp.float32)
        return _

    lax.fori_loop(0, B, _zrow, None)
    zoff = 0
    while zoff < ZR:
        sz = min(B, ZR - zoff)
        pltpu.sync_copy(buf.at[0, pl.ds(0, sz)],
                        agg.at[pl.ds(sid * ZR + zoff, sz)])
        zoff += sz
    plsc.subcore_barrier()

    sis = (si0, si1)
    sgs = (sg0, sg1)
    pltpu.make_async_copy(rc_hbm.at[wid, 0], rcv.at[0], si0).start()
    pltpu.make_async_copy(rc_hbm.at[wid, 1], rcv.at[1], si1).start()
    pltpu.make_async_copy(rc_hbm.at[wid, 0], rcv.at[0], si0).wait()
    pltpu.make_async_copy(xs_hbm.at[rcv.at[0, 0]], buf.at[0], sg0).start()

    def _step(jj, _):
        for b in range(2):
            j = jj * 2 + b
            nb = 1 - b

            @pl.when(j + 1 < NCHUNK)
            def _start_next_gather():
                pltpu.make_async_copy(rc_hbm.at[wid, j + 1], rcv.at[nb],
                                      sis[nb]).wait()
                pltpu.make_async_copy(xs_hbm.at[rcv.at[nb, 0]], buf.at[nb],
                                      sgs[nb]).start()

            pltpu.make_async_copy(xs_hbm.at[rcv.at[b, 0]], buf.at[b],
                                  sgs[b]).wait()
            pltpu.sync_copy(buf.at[b], agg.at[rcv.at[b, 1]], add=True)

            @pl.when(j + 2 < NCHUNK)
            def _start_next_idx():
                pltpu.make_async_copy(rc_hbm.at[wid, j + 2], rcv.at[b],
                                      sis[b]).start()

        return _

    lax.fori_loop(0, NCHUNK // 2, _step, None)
    plsc.subcore_barrier()

    def _copy_out(nrows):
        coff = 0
        while coff < nrows:
            sz = min(B, nrows - coff)
            base = sid * CR + coff
            pltpu.sync_copy(agg.at[pl.ds(base, sz)], buf.at[0, pl.ds(0, sz)])
            pltpu.sync_copy(buf.at[0, pl.ds(0, sz)],
                            aggp_hbm.at[cid, pl.ds(base, sz)])
            coff += sz

    @pl.when(sid < NS - 1)
    def _():
        _copy_out(CR)

    @pl.when(sid == NS - 1)
    def _():
        _copy_out(N - (NS - 1) * CR)


def _tc_scale_body(dp_ref, x_ref, xs_ref, rn_ref):
    dp = dp_ref[0]
    ones = jnp.ones((NW, 1), jnp.float32)
    dcol = lax.dot_general(dp, ones, (((0,), (0,)), ((), ())),
                           preferred_element_type=jnp.float32)
    rn = jnp.where(dcol > 0.0, lax.rsqrt(dcol), 0.0)
    rn_ref[...] = rn
    xs_ref[...] = x_ref[...] * rn


def _tc_dense_body(aggp_ref, rn_ref, x_ref, e_ref, w_ref, out_ref):
    s = aggp_ref[0] + aggp_ref[1]
    h1 = s * rn_ref[...]
    x = x_ref[...]
    acc = x
    for k in range(K):
        yk = (jnp.dot(h1, w_ref[k, :F, :],
                      preferred_element_type=jnp.float32)
              + jnp.dot(x, w_ref[k, F:, :],
                        preferred_element_type=jnp.float32))
        acc = acc + e_ref[:, k:k + 1] * yk
    out_ref[...] = acc


_R = 1000


def kernel(x, adj, e, W):
    row = adj[0]
    col = adj[1]
    pad = EPAD - E
    row_p = jnp.concatenate(
        [row, jnp.zeros((pad,), jnp.int32)]).reshape(NW, NCHUNK, B)
    col_p = jnp.concatenate(
        [col, jnp.full((pad,), N, jnp.int32)]).reshape(NW, NCHUNK, B)
    rc_p = jnp.stack([row_p, col_p], axis=2)

    dpart = _sc_degree(col_p)
    dpart_t = (dpart.reshape(NW, DN)[:, :N]
               .reshape(NW, N // _R, _R).transpose(1, 0, 2))

    xs, rn = pl.pallas_call(
        _tc_scale_body,
        grid=(N // _R,),
        in_specs=[
            pl.BlockSpec((1, NW, _R), lambda i: (i, 0, 0)),
            pl.BlockSpec((_R, F), lambda i: (i, 0)),
        ],
        out_specs=[
            pl.BlockSpec((_R, F), lambda i: (i, 0)),
            pl.BlockSpec((_R, 1), lambda i: (i, 0)),
        ],
        out_shape=[
            jax.ShapeDtypeStruct((N, F), jnp.float32),
            jax.ShapeDtypeStruct((N, 1), jnp.float32),
        ],
    )(dpart_t, x)

    aggp = _sc_spmm(rc_p, xs)

    out = pl.pallas_call(
        _tc_dense_body,
        grid=(N // _R,),
        in_specs=[
            pl.BlockSpec((NC, _R, F), lambda i: (0, i, 0)),
            pl.BlockSpec((_R, 1), lambda i: (i, 0)),
            pl.BlockSpec((_R, F), lambda i: (i, 0)),
            pl.BlockSpec((_R, K), lambda i: (i, 0)),
            pl.BlockSpec((K, 2 * F, F), lambda i: (0, 0, 0)),
        ],
        out_specs=pl.BlockSpec((_R, F), lambda i: (i, 0)),
        out_shape=jax.ShapeDtypeStruct((N, F), jnp.float32),
    )(aggp, rn, x, e, W)
    return out
